# baseline (speedup 1.0000x reference)
import functools

import jax
import jax.numpy as jnp
from jax import lax
from jax.experimental import pallas as pl
from jax.experimental.pallas import tpu as pltpu
from jax.experimental.pallas import tpu_sc as plsc

VOCAB = 1000000
D = 64
NCLS = 16
B = 16384
SEQ = 20
HID = 32

NC = 2
NS = 16
NW = NC * NS

NFULL = VOCAB // 128
FULL_PER_W = NFULL // NW
FULL_REM = NFULL - NW * FULL_PER_W
TAIL0 = NFULL * 128
TAILW = VOCAB - TAIL0

BPW = B // NW
CHUNK_B = 16
NCHUNK = BPW // CHUNK_B
CHUNK_ROWS = CHUNK_B * SEQ
OPW = BPW // 2


TRB = 32768
TRGRID = -(-VOCAB // TRB)


def _tr_body(embt_ref, o_ref):
    eye = jnp.eye(D, dtype=jnp.float32)
    t = lax.dot_general(embt_ref[...], eye, (((0,), (0,)), ((), ())),
                        preferred_element_type=jnp.float32)
    o_ref[...] = jnp.concatenate([t, jnp.zeros_like(t)], axis=1)


_tc_transpose = pl.pallas_call(
    _tr_body,
    grid=(TRGRID,),
    in_specs=[pl.BlockSpec((D, TRB), lambda i: (0, i))],
    out_specs=pl.BlockSpec((TRB, 2 * D), lambda i: (i, 0)),
    out_shape=jax.ShapeDtypeStruct((VOCAB, 2 * D), jnp.float32),
)


GPLAN = ((0, 128), (128, 128), (256, 64))


def _sc_body(xt_hbm, table_hbm, out_hbm, idx_v, idx2_v, rows_v, out_v,
             sem0, sem1):
    cid = lax.axis_index("c")
    sid = lax.axis_index("s")
    wid = sid * NC + cid

    pltpu.sync_copy(xt_hbm.at[:, pl.ds(wid * BPW, BPW)], idx_v)

    def issue(c, par, sem):
        def j_body(j, carry):
            idx2_v[par, pl.ds(j * CHUNK_B, CHUNK_B)] = (
                idx_v[j, pl.ds(c * CHUNK_B, CHUNK_B)])
            return carry
        lax.fori_loop(0, SEQ, j_body, 0)
        for off, n in GPLAN:
            pltpu.async_copy(
                table_hbm.at[idx2_v.at[par, pl.ds(off, n)]],
                rows_v.at[par, pl.ds(off, n)],
                sem,
            )

    def drain(par, sem):
        for off, n in GPLAN:
            pltpu.make_async_copy(
                table_hbm.at[pl.ds(0, n), :],
                rows_v.at[par, pl.ds(off, n)],
                sem,
            ).wait()

    def pool(c, par):
        def b_body(bb, carry):
            for half in range(2):
                bag = 2 * bb + half
                for dd in range(D // 16):
                    acc = rows_v[par, bag, pl.ds(dd * 16, 16)]
                    for j in range(1, SEQ):
                        acc = acc + rows_v[par, j * CHUNK_B + bag,
                                           pl.ds(dd * 16, 16)]
                    out_v[c * (CHUNK_B // 2) + bb,
                          pl.ds(half * D + dd * 16, 16)] = acc
            return carry
        lax.fori_loop(0, CHUNK_B // 2, b_body, 0)

    issue(0, 0, sem0)

    def outer(k, carry):
        c0 = 2 * k
        c1 = 2 * k + 1
        issue(c1, 1, sem1)
        drain(0, sem0)
        pool(c0, 0)

        @pl.when(c1 + 1 < NCHUNK)
        def _():
            issue(c1 + 1, 0, sem0)

        drain(1, sem1)
        pool(c1, 1)
        return carry

    lax.fori_loop(0, NCHUNK // 2, outer, 0)

    pltpu.sync_copy(out_v, out_hbm.at[pl.ds(wid * OPW, OPW), :])


_sc_pool = functools.partial(
    pl.kernel,
    out_type=jax.ShapeDtypeStruct((B // 2, 2 * D), jnp.float32),
    mesh=plsc.VectorSubcoreMesh(core_axis_name="c", subcore_axis_name="s"),
    scratch_types=[
        pltpu.VMEM((SEQ, BPW), jnp.int32),
        pltpu.VMEM((2, CHUNK_ROWS), jnp.int32),
        pltpu.VMEM((2, CHUNK_ROWS, 2 * D), jnp.float32),
        pltpu.VMEM((OPW, 2 * D), jnp.float32),
        pltpu.SemaphoreType.DMA,
        pltpu.SemaphoreType.DMA,
    ],
)(_sc_body)


MB = 2048


def _mlp_body(p_ref, w1_ref, b1_ref, w2_ref, b2_ref, o_ref):
    p = p_ref[...] * (1.0 / SEQ)
    h = lax.dot_general(p, w1_ref[...], (((1,), (1,)), ((), ())),
                        preferred_element_type=jnp.float32)
    h = h + b1_ref[...]
    o = lax.dot_general(h, w2_ref[...], (((1,), (1,)), ((), ())),
                        preferred_element_type=jnp.float32)
    o_ref[...] = o + b2_ref[...]


_mlp = pl.pallas_call(
    _mlp_body,
    grid=(B // MB,),
    in_specs=[
        pl.BlockSpec((MB, D), lambda i: (i, 0)),
        pl.BlockSpec((HID, D), lambda i: (0, 0)),
        pl.BlockSpec((1, HID), lambda i: (0, 0)),
        pl.BlockSpec((NCLS, HID), lambda i: (0, 0)),
        pl.BlockSpec((1, NCLS), lambda i: (0, 0)),
    ],
    out_specs=pl.BlockSpec((MB, NCLS), lambda i: (i, 0)),
    out_shape=jax.ShapeDtypeStruct((B, NCLS), jnp.float32),
)


def kernel(x, emb_table, W1, b1, W2, b2):
    xt = x.astype(jnp.int32).T
    table_pad = _tc_transpose(emb_table.T)
    pooled_sum = _sc_pool(xt, table_pad).reshape(B, D)
    return _mlp(pooled_sum, W1, b1.reshape(1, HID), W2, b2.reshape(1, NCLS))


# speedup vs baseline: 1.0050x; 1.0050x over previous
import functools

import jax
import jax.numpy as jnp
from jax import lax
from jax.experimental import pallas as pl
from jax.experimental.pallas import tpu as pltpu
from jax.experimental.pallas import tpu_sc as plsc

VOCAB = 1000000
D = 64
NCLS = 16
B = 16384
SEQ = 20
HID = 32

NC = 2
NS = 16
NW = NC * NS

NFULL = VOCAB // 128
FULL_PER_W = NFULL // NW
FULL_REM = NFULL - NW * FULL_PER_W
TAIL0 = NFULL * 128
TAILW = VOCAB - TAIL0

BPW = B // NW
CHUNK_B = 16
NCHUNK = BPW // CHUNK_B
CHUNK_ROWS = CHUNK_B * SEQ
OPW = BPW // 2


TRB = 32768
TRGRID = -(-VOCAB // TRB)


def _tr_body(embt_ref, o_ref):
    eye = jnp.eye(D, dtype=jnp.float32)
    t = lax.dot_general(embt_ref[...], eye, (((0,), (0,)), ((), ())),
                        preferred_element_type=jnp.float32)
    o_ref[...] = jnp.concatenate([t, jnp.zeros_like(t)], axis=1)


_tc_transpose = pl.pallas_call(
    _tr_body,
    grid=(TRGRID,),
    in_specs=[pl.BlockSpec((D, TRB), lambda i: (0, i))],
    out_specs=pl.BlockSpec((TRB, 2 * D), lambda i: (i, 0)),
    out_shape=jax.ShapeDtypeStruct((VOCAB, 2 * D), jnp.float32),
)


GPLAN = ((0, 128), (128, 128), (256, 64))


def _sc_body(xt_hbm, table_hbm, out_hbm, idx_v, idx2_v, rows_v, out_v,
             sem0, sem1):
    cid = lax.axis_index("c")
    sid = lax.axis_index("s")
    wid = sid * NC + cid

    pltpu.sync_copy(xt_hbm.at[:, pl.ds(wid * BPW, BPW)], idx_v)

    def issue(c, par, sem):
        def j_body(j, carry):
            idx2_v[par, pl.ds(j * CHUNK_B, CHUNK_B)] = (
                idx_v[j, pl.ds(c * CHUNK_B, CHUNK_B)])
            return carry
        lax.fori_loop(0, SEQ, j_body, 0)
        for off, n in GPLAN:
            pltpu.async_copy(
                table_hbm.at[idx2_v.at[par, pl.ds(off, n)]],
                rows_v.at[par, pl.ds(off, n)],
                sem,
            )

    def drain(par, sem):
        for off, n in GPLAN:
            pltpu.make_async_copy(
                table_hbm.at[pl.ds(0, n), :],
                rows_v.at[par, pl.ds(off, n)],
                sem,
            ).wait()

    def pool(c, par):
        def b_body(bb, carry):
            for half in range(2):
                bag = 2 * bb + half
                for dd in range(D // 16):
                    acc = rows_v[par, bag, pl.ds(dd * 16, 16)]
                    for j in range(1, SEQ):
                        acc = acc + rows_v[par, j * CHUNK_B + bag,
                                           pl.ds(dd * 16, 16)]
                    out_v[c * (CHUNK_B // 2) + bb,
                          pl.ds(half * D + dd * 16, 16)] = acc
            return carry
        lax.fori_loop(0, CHUNK_B // 2, b_body, 0)

    issue(0, 0, sem0)

    def outer(k, carry):
        c0 = 2 * k
        c1 = 2 * k + 1
        issue(c1, 1, sem1)
        drain(0, sem0)
        pool(c0, 0)

        @pl.when(c1 + 1 < NCHUNK)
        def _():
            issue(c1 + 1, 0, sem0)

        drain(1, sem1)
        pool(c1, 1)
        return carry

    lax.fori_loop(0, NCHUNK // 2, outer, 0)

    pltpu.sync_copy(out_v, out_hbm.at[pl.ds(wid * OPW, OPW), :])


_sc_pool = functools.partial(
    pl.kernel,
    out_type=jax.ShapeDtypeStruct((B // 2, 2 * D), jnp.float32),
    mesh=plsc.VectorSubcoreMesh(core_axis_name="c", subcore_axis_name="s"),
    scratch_types=[
        pltpu.VMEM((SEQ, BPW), jnp.int32),
        pltpu.VMEM((2, CHUNK_ROWS), jnp.int32),
        pltpu.VMEM((2, CHUNK_ROWS, 2 * D), jnp.float32),
        pltpu.VMEM((OPW, 2 * D), jnp.float32),
        pltpu.SemaphoreType.DMA,
        pltpu.SemaphoreType.DMA,
    ],
)(_sc_body)


MB2 = 1024


def _mlp_body(p_ref, w1_ref, b1_ref, w2_ref, b2_ref, o_ref):
    p2 = p_ref[...] * (1.0 / SEQ)
    outs = []
    for half in range(2):
        p = p2[:, half * D:(half + 1) * D]
        h = lax.dot_general(p, w1_ref[...], (((1,), (1,)), ((), ())),
                            preferred_element_type=jnp.float32)
        h = h + b1_ref[...]
        o = lax.dot_general(h, w2_ref[...], (((1,), (1,)), ((), ())),
                            preferred_element_type=jnp.float32)
        outs.append(o + b2_ref[...])
    o_ref[...] = jnp.concatenate(outs, axis=1)


_mlp = pl.pallas_call(
    _mlp_body,
    grid=(B // 2 // MB2,),
    in_specs=[
        pl.BlockSpec((MB2, 2 * D), lambda i: (i, 0)),
        pl.BlockSpec((HID, D), lambda i: (0, 0)),
        pl.BlockSpec((1, HID), lambda i: (0, 0)),
        pl.BlockSpec((NCLS, HID), lambda i: (0, 0)),
        pl.BlockSpec((1, NCLS), lambda i: (0, 0)),
    ],
    out_specs=pl.BlockSpec((MB2, 2 * NCLS), lambda i: (i, 0)),
    out_shape=jax.ShapeDtypeStruct((B // 2, 2 * NCLS), jnp.float32),
)


def kernel(x, emb_table, W1, b1, W2, b2):
    xt = x.astype(jnp.int32).T
    table_pad = _tc_transpose(emb_table.T)
    pooled_packed = _sc_pool(xt, table_pad)
    out2 = _mlp(pooled_packed, W1, b1.reshape(1, HID), W2,
                b2.reshape(1, NCLS))
    return out2.reshape(B, NCLS)


# speedup vs baseline: 1.0123x; 1.0073x over previous
import functools

import jax
import jax.numpy as jnp
from jax import lax
from jax.experimental import pallas as pl
from jax.experimental.pallas import tpu as pltpu
from jax.experimental.pallas import tpu_sc as plsc

VOCAB = 1000000
D = 64
NCLS = 16
B = 16384
SEQ = 20
HID = 32

NC = 2
NS = 16
NW = NC * NS

NFULL = VOCAB // 128
FULL_PER_W = NFULL // NW
FULL_REM = NFULL - NW * FULL_PER_W
TAIL0 = NFULL * 128
TAILW = VOCAB - TAIL0

BPW = B // NW
CHUNK_B = 16
NCHUNK = BPW // CHUNK_B
CHUNK_ROWS = CHUNK_B * SEQ
OPW = BPW // 2


TRB = 32768
TRGRID = -(-VOCAB // TRB)


def _tr_body(embt_ref, o_ref):
    eye = jnp.eye(D, dtype=jnp.float32)
    t = lax.dot_general(embt_ref[...], eye, (((0,), (0,)), ((), ())),
                        preferred_element_type=jnp.float32)
    o_ref[...] = jnp.concatenate([t, jnp.zeros_like(t)], axis=1)


_tc_transpose = pl.pallas_call(
    _tr_body,
    grid=(TRGRID,),
    in_specs=[pl.BlockSpec((D, TRB), lambda i: (0, i))],
    out_specs=pl.BlockSpec((TRB, 2 * D), lambda i: (i, 0)),
    out_shape=jax.ShapeDtypeStruct((VOCAB, 2 * D), jnp.float32),
)


GPLAN = ((0, 128), (128, 128), (256, 64))


def _sc_body(xt_hbm, table_hbm, out_hbm, idx_v, idx2_v, rows_v, out_v,
             sem0, sem1):
    cid = lax.axis_index("c")
    sid = lax.axis_index("s")
    wid = sid * NC + cid

    pltpu.sync_copy(xt_hbm.at[:, pl.ds(wid * BPW, BPW)], idx_v)

    def issue(c, par, sem):
        def j_body(j, carry):
            idx2_v[par, pl.ds(j * CHUNK_B, CHUNK_B)] = (
                idx_v[j, pl.ds(c * CHUNK_B, CHUNK_B)])
            return carry
        lax.fori_loop(0, SEQ, j_body, 0)
        for off, n in GPLAN:
            pltpu.async_copy(
                table_hbm.at[idx2_v.at[par, pl.ds(off, n)]],
                rows_v.at[par, pl.ds(off, n)],
                sem,
            )

    def drain(par, sem):
        for off, n in GPLAN:
            pltpu.make_async_copy(
                table_hbm.at[pl.ds(0, n), :],
                rows_v.at[par, pl.ds(off, n)],
                sem,
            ).wait()

    def pool(c, par):
        def b_body(bb, carry):
            for half in range(2):
                bag = 2 * bb + half
                for dd in range(D // 16):
                    accs = [rows_v[par, j * CHUNK_B + bag,
                                   pl.ds(dd * 16, 16)] for j in range(4)]
                    for j in range(4, SEQ):
                        accs[j % 4] = accs[j % 4] + rows_v[
                            par, j * CHUNK_B + bag, pl.ds(dd * 16, 16)]
                    acc = (accs[0] + accs[1]) + (accs[2] + accs[3])
                    out_v[c * (CHUNK_B // 2) + bb,
                          pl.ds(half * D + dd * 16, 16)] = acc
            return carry
        lax.fori_loop(0, CHUNK_B // 2, b_body, 0)

    issue(0, 0, sem0)

    def outer(k, carry):
        c0 = 2 * k
        c1 = 2 * k + 1
        issue(c1, 1, sem1)
        drain(0, sem0)
        pool(c0, 0)

        @pl.when(c1 + 1 < NCHUNK)
        def _():
            issue(c1 + 1, 0, sem0)

        drain(1, sem1)
        pool(c1, 1)
        return carry

    lax.fori_loop(0, NCHUNK // 2, outer, 0)

    pltpu.sync_copy(out_v, out_hbm.at[pl.ds(wid * OPW, OPW), :])


_sc_pool = functools.partial(
    pl.kernel,
    out_type=jax.ShapeDtypeStruct((B // 2, 2 * D), jnp.float32),
    mesh=plsc.VectorSubcoreMesh(core_axis_name="c", subcore_axis_name="s"),
    scratch_types=[
        pltpu.VMEM((SEQ, BPW), jnp.int32),
        pltpu.VMEM((2, CHUNK_ROWS), jnp.int32),
        pltpu.VMEM((2, CHUNK_ROWS, 2 * D), jnp.float32),
        pltpu.VMEM((OPW, 2 * D), jnp.float32),
        pltpu.SemaphoreType.DMA,
        pltpu.SemaphoreType.DMA,
    ],
)(_sc_body)


MB2 = 1024


def _mlp_body(p_ref, w1_ref, b1_ref, w2_ref, b2_ref, o_ref):
    p2 = p_ref[...] * (1.0 / SEQ)
    outs = []
    for half in range(2):
        p = p2[:, half * D:(half + 1) * D]
        h = lax.dot_general(p, w1_ref[...], (((1,), (1,)), ((), ())),
                            preferred_element_type=jnp.float32)
        h = h + b1_ref[...]
        o = lax.dot_general(h, w2_ref[...], (((1,), (1,)), ((), ())),
                            preferred_element_type=jnp.float32)
        outs.append(o + b2_ref[...])
    o_ref[...] = jnp.concatenate(outs, axis=1)


_mlp = pl.pallas_call(
    _mlp_body,
    grid=(B // 2 // MB2,),
    in_specs=[
        pl.BlockSpec((MB2, 2 * D), lambda i: (i, 0)),
        pl.BlockSpec((HID, D), lambda i: (0, 0)),
        pl.BlockSpec((1, HID), lambda i: (0, 0)),
        pl.BlockSpec((NCLS, HID), lambda i: (0, 0)),
        pl.BlockSpec((1, NCLS), lambda i: (0, 0)),
    ],
    out_specs=pl.BlockSpec((MB2, 2 * NCLS), lambda i: (i, 0)),
    out_shape=jax.ShapeDtypeStruct((B // 2, 2 * NCLS), jnp.float32),
)


def kernel(x, emb_table, W1, b1, W2, b2):
    xt = x.astype(jnp.int32).T
    table_pad = _tc_transpose(emb_table.T)
    pooled_packed = _sc_pool(xt, table_pad)
    out2 = _mlp(pooled_packed, W1, b1.reshape(1, HID), W2,
                b2.reshape(1, NCLS))
    return out2.reshape(B, NCLS)
